# baseline (speedup 1.0000x reference)
import jax
import jax.numpy as jnp
from jax import lax
from jax.experimental import pallas as pl
from jax.experimental.pallas import tpu as pltpu
from jax.experimental.pallas import tpu_sc as plsc

_TOPK = 2
_NID = 2
_NMLP = 6
_NE = 8
_D = 768
_T = 8192
_P = _T * _TOPK
_NW = 32
_PPW = _P // _NW
_TPW = _T // _NW
_TB2 = 512
_CAP = _P + _NE * _TB2
_NBLK = _CAP // _TB2
_NBLKP = 48
_CH = 64
_LANE = 16


def _wid():
    return lax.axis_index("s") * 2 + lax.axis_index("c")


def _hist_body(si_ref, cnt_ref, sib, cbuf, sem):
    w = _wid()
    pltpu.sync_copy(si_ref.at[w], sib)
    lane = lax.iota(jnp.int32, _LANE)
    cnt = jnp.zeros((_LANE,), jnp.int32)
    for j in range(_PPW // _CH):
        for c in range(_CH // _LANE):
            v = sib[j, pl.ds(c * _LANE, _LANE)]
            for e in range(_NE):
                pc = jnp.sum(jnp.where(v == e, 1, 0).astype(jnp.int32))
                cnt = cnt + jnp.where(lane == e, pc, 0)
    cbuf[...] = cnt
    pltpu.sync_copy(cbuf, cnt_ref.at[w])


def _dispatch_body(si_ref, sw_ref, cnt_ref, x_ref,
                   xg_ref, swo_ref, pos_ref, blk_ref,
                   sib, swb, posb, tokb, cntsb, blkb, rbuf, sem):
    w = _wid()
    lane = lax.iota(jnp.int32, _LANE)
    pltpu.sync_copy(si_ref.at[w], sib)
    pltpu.sync_copy(sw_ref.at[w], swb)
    pltpu.sync_copy(cnt_ref, cntsb)

    tot = jnp.zeros((_LANE,), jnp.int32)
    run = jnp.zeros((_LANE,), jnp.int32)
    wsplat = jnp.full((_LANE,), w, jnp.int32)
    for t in range(_NW):
        row = cntsb[t, :]
        tot = tot + row
        run = run + jnp.where(wsplat > t, row, 0)
    pc = ((tot + (_TB2 - 1)) >> 9) << 9
    base = plsc.cumsum(pc) - pc
    start = base + run
    ends = plsc.cumsum(pc)

    for g in range(_NBLKP // _LANE):
        bstart = (lane + g * _LANE) * _TB2
        acc = jnp.zeros((_LANE,), jnp.int32)
        for e in range(_NE):
            end_e = jnp.sum(jnp.where(lane == e, ends, 0))
            acc = acc + jnp.where(bstart >= end_e, 1, 0)
        blkb[pl.ds(g * _LANE, _LANE)] = acc

    @pl.when(w == 0)
    def _():
        pltpu.sync_copy(blkb, blk_ref)

    for e in range(_NE):
        s_e = jnp.sum(jnp.where(lane == e, start, 0))
        rk = jnp.int32(0)
        for j in range(_PPW // _CH):
            for c in range(_CH // _LANE):
                v = sib[j, pl.ds(c * _LANE, _LANE)]
                m = v == e
                mi = jnp.where(m, 1, 0).astype(jnp.int32)
                pr = plsc.cumsum(mi) - mi
                posv = pr + (s_e + rk)
                old = posb[j, pl.ds(c * _LANE, _LANE)]
                posb[j, pl.ds(c * _LANE, _LANE)] = jnp.where(m, posv, old)
                rk = rk + jnp.sum(mi)

    pbase = w * _PPW
    for j in range(_PPW // _CH):
        for c in range(_CH // _LANE):
            p = pbase + j * _CH + c * _LANE + lane
            tokb[j, pl.ds(c * _LANE, _LANE)] = p >> 1

    pltpu.sync_copy(posb, pos_ref.at[w])

    for j in range(_PPW // _CH):
        pltpu.async_copy(swb.at[j], swo_ref.at[posb.at[j]], sem).wait()

    for j in range(_PPW // _CH):
        pltpu.async_copy(x_ref.at[tokb.at[j]], rbuf, sem).wait()
        pltpu.async_copy(rbuf, xg_ref.at[posb.at[j]], sem).wait()


def _mlp_body(be_ref, xg_ref, sw_ref, g_ref, b_ref, w1_ref, b1_ref, w2_ref,
              b2_ref, o_ref):
    i = pl.program_id(0)
    e = be_ref[i]
    xr = xg_ref[...]
    sw = sw_ref[...]

    @pl.when(jnp.logical_or(e < _NID, e >= _NE))
    def _():
        o_ref[...] = sw * xr

    @pl.when(jnp.logical_and(e >= _NID, e < _NE))
    def _():
        mu = jnp.mean(xr, axis=1, keepdims=True)
        xc = xr - mu
        var = jnp.mean(xc * xc, axis=1, keepdims=True)
        z = xc * lax.rsqrt(var + 1e-5)
        zm = z * g_ref[0] + b_ref[0]
        h = jnp.dot(zm, w1_ref[0], preferred_element_type=jnp.float32) + b1_ref[0]
        gh = 0.5 * h * (1.0 + lax.erf(h * 0.7071067811865476))
        out = jnp.dot(gh, w2_ref[0], preferred_element_type=jnp.float32) + b2_ref[0]
        o_ref[...] = sw * (xr + out)


def _combine_body(pos_ref, po_ref, y_ref, posb, pA, pB, ra, rb, obuf, sem):
    w = _wid()
    lane = lax.iota(jnp.int32, _LANE)
    pltpu.sync_copy(pos_ref.at[w], posb)
    for r in range(_PPW // _CH):
        for j in range(2):
            idx = (j * _LANE + lane) * 2
            a = plsc.load_gather(posb, [jnp.full((_LANE,), r, jnp.int32), idx])
            b = plsc.load_gather(posb, [jnp.full((_LANE,), r, jnp.int32), idx + 1])
            pA[r, pl.ds(j * _LANE, _LANE)] = a
            pB[r, pl.ds(j * _LANE, _LANE)] = b
    nch = _PPW // _CH
    tpc = _CH // 2
    for r in range(nch):
        pltpu.async_copy(po_ref.at[pA.at[r]], ra, sem).wait()
        pltpu.async_copy(po_ref.at[pB.at[r]], rb, sem).wait()

        def body(t, carry):
            for c in range(_D // _LANE):
                av = ra[t, pl.ds(c * _LANE, _LANE)]
                bv = rb[t, pl.ds(c * _LANE, _LANE)]
                obuf[t, pl.ds(c * _LANE, _LANE)] = av + bv
            return carry

        lax.fori_loop(0, tpc, body, jnp.int32(0))
        pltpu.sync_copy(obuf, y_ref.at[w, pl.ds(r * tpc, tpc)])


def _sc_mesh():
    return plsc.VectorSubcoreMesh(core_axis_name="c", subcore_axis_name="s")


_SC_PARAMS = pltpu.CompilerParams(needs_layout_passes=False)


def kernel(x, selected_indices, selected_weights, ln_g, ln_b, W1, b1, W2, b2):
    B, N, D = x.shape
    xf = x.reshape(_T, D)
    si3 = selected_indices.astype(jnp.int32).reshape(_NW, _PPW // _CH, _CH)
    sw3 = selected_weights.reshape(_NW, _PPW // _CH, _CH)

    counts = pl.kernel(
        _hist_body,
        out_type=jax.ShapeDtypeStruct((_NW, _LANE), jnp.int32),
        mesh=_sc_mesh(),
        scratch_types=[
            pltpu.VMEM((_PPW // _CH, _CH), jnp.int32),
            pltpu.VMEM((_LANE,), jnp.int32),
            pltpu.SemaphoreType.DMA,
        ],
        compiler_params=_SC_PARAMS,
    )(si3)

    xg, swo, pos3, blk = pl.kernel(
        _dispatch_body,
        out_type=(
            jax.ShapeDtypeStruct((_CAP, _D), jnp.float32),
            jax.ShapeDtypeStruct((_CAP,), jnp.float32),
            jax.ShapeDtypeStruct((_NW, _PPW // _CH, _CH), jnp.int32),
            jax.ShapeDtypeStruct((_NBLKP,), jnp.int32),
        ),
        mesh=_sc_mesh(),
        scratch_types=[
            pltpu.VMEM((_PPW // _CH, _CH), jnp.int32),
            pltpu.VMEM((_PPW // _CH, _CH), jnp.float32),
            pltpu.VMEM((_PPW // _CH, _CH), jnp.int32),
            pltpu.VMEM((_PPW // _CH, _CH), jnp.int32),
            pltpu.VMEM((_NW, _LANE), jnp.int32),
            pltpu.VMEM((_NBLKP,), jnp.int32),
            pltpu.VMEM((_CH, _D), jnp.float32),
            pltpu.SemaphoreType.DMA,
        ],
        compiler_params=_SC_PARAMS,
    )(si3, sw3, counts, xf)

    gg = ln_g[:, None, :]
    bb = ln_b[:, None, :]
    b1r = b1[:, None, :]
    b2r = b2[:, None, :]
    swo2 = swo.reshape(_CAP, 1)

    def em(i, be):
        return jnp.clip(be[i] - _NID, 0, _NMLP - 1)

    po = pl.pallas_call(
        _mlp_body,
        grid_spec=pltpu.PrefetchScalarGridSpec(
            num_scalar_prefetch=1,
            grid=(_NBLK,),
            in_specs=[
                pl.BlockSpec((_TB2, _D), lambda i, be: (i, 0)),
                pl.BlockSpec((_TB2, 1), lambda i, be: (i, 0)),
                pl.BlockSpec((1, 1, _D), lambda i, be: (em(i, be), 0, 0)),
                pl.BlockSpec((1, 1, _D), lambda i, be: (em(i, be), 0, 0)),
                pl.BlockSpec((1, _D, _D), lambda i, be: (em(i, be), 0, 0)),
                pl.BlockSpec((1, 1, _D), lambda i, be: (em(i, be), 0, 0)),
                pl.BlockSpec((1, _D, _D), lambda i, be: (em(i, be), 0, 0)),
                pl.BlockSpec((1, 1, _D), lambda i, be: (em(i, be), 0, 0)),
            ],
            out_specs=pl.BlockSpec((_TB2, _D), lambda i, be: (i, 0)),
        ),
        out_shape=jax.ShapeDtypeStruct((_CAP, _D), jnp.float32),
        compiler_params=pltpu.CompilerParams(
            dimension_semantics=("arbitrary",),
        ),
    )(blk, xg, swo2, gg, bb, W1, b1r, W2, b2r)

    y = pl.kernel(
        _combine_body,
        out_type=jax.ShapeDtypeStruct((_NW, _TPW, _D), jnp.float32),
        mesh=_sc_mesh(),
        scratch_types=[
            pltpu.VMEM((_PPW // _CH, _CH), jnp.int32),
            pltpu.VMEM((_PPW // _CH, _CH // 2), jnp.int32),
            pltpu.VMEM((_PPW // _CH, _CH // 2), jnp.int32),
            pltpu.VMEM((_CH // 2, _D), jnp.float32),
            pltpu.VMEM((_CH // 2, _D), jnp.float32),
            pltpu.VMEM((_CH // 2, _D), jnp.float32),
            pltpu.SemaphoreType.DMA,
        ],
        compiler_params=_SC_PARAMS,
    )(pos3, po)

    return y.reshape(B, N, D)


# speedup vs baseline: 1.7713x; 1.7713x over previous
import jax
import jax.numpy as jnp
from jax.experimental import pallas as pl
from jax.experimental.pallas import tpu as pltpu

_TOPK = 2
_NID = 2
_NMLP = 6
_TB = 512


def _moe_body(si_ref, sw_ref, x_ref, g_ref, b_ref, w1_ref, b1_ref, w2_ref,
              b2_ref, o_ref):
    x = x_ref[...]
    mu = jnp.mean(x, axis=1, keepdims=True)
    xc = x - mu
    var = jnp.mean(xc * xc, axis=1, keepdims=True)
    z = xc * jax.lax.rsqrt(var + 1e-5)

    si = si_ref[...]
    sw = sw_ref[...]
    parts = []
    wms = []
    for m in range(_NMLP):
        wm = jnp.sum(jnp.where(si == (m + _NID), sw, 0.0), axis=1, keepdims=True)
        zm = z * g_ref[m] + b_ref[m]
        h = jnp.dot(zm, w1_ref[m], preferred_element_type=jnp.float32) + b1_ref[m]
        g = 0.5 * h * (1.0 + jax.lax.erf(h * 0.7071067811865476))
        parts.append(wm * g)
        wms.append(wm)
    gall = jnp.concatenate(parts, axis=1)
    wmat = jnp.concatenate(wms, axis=1)
    acc = (jnp.sum(sw, axis=1, keepdims=True) * x
           + jnp.dot(gall, w2_ref[...], preferred_element_type=jnp.float32)
           + jnp.dot(wmat, b2_ref[...], preferred_element_type=jnp.float32))
    o_ref[...] = acc


def kernel(x, selected_indices, selected_weights, ln_g, ln_b, W1, b1, W2, b2):
    B, N, D = x.shape
    T = B * N
    xf = x.reshape(T, D)
    si = selected_indices.reshape(T, _TOPK)
    sw = selected_weights.reshape(T, _TOPK)
    gg = ln_g[:, None, :]
    bb = ln_b[:, None, :]
    b1r = b1[:, None, :]
    W2s = W2.reshape(_NMLP * D, D)

    grid = (T // _TB,)
    out = pl.pallas_call(
        _moe_body,
        grid=grid,
        in_specs=[
            pl.BlockSpec((_TB, _TOPK), lambda i: (i, 0)),
            pl.BlockSpec((_TB, _TOPK), lambda i: (i, 0)),
            pl.BlockSpec((_TB, D), lambda i: (i, 0)),
            pl.BlockSpec((_NMLP, 1, D), lambda i: (0, 0, 0)),
            pl.BlockSpec((_NMLP, 1, D), lambda i: (0, 0, 0)),
            pl.BlockSpec((_NMLP, D, D), lambda i: (0, 0, 0)),
            pl.BlockSpec((_NMLP, 1, D), lambda i: (0, 0, 0)),
            pl.BlockSpec((_NMLP * D, D), lambda i: (0, 0)),
            pl.BlockSpec((_NMLP, D), lambda i: (0, 0)),
        ],
        out_specs=pl.BlockSpec((_TB, D), lambda i: (i, 0)),
        out_shape=jax.ShapeDtypeStruct((T, D), jnp.float32),
        compiler_params=pltpu.CompilerParams(
            dimension_semantics=("arbitrary",),
        ),
    )(si, sw, xf, gg, bb, W1, b1r, W2s, b2)
    return out.reshape(B, N, D)
